# Initial kernel scaffold; baseline (speedup 1.0000x reference)
import jax
import jax.numpy as jnp
from jax.experimental import pallas as pl


def kernel(x, edge_index, W1_0, b1_0, W2_0, b2_0, W1_1, b1_1, W2_1, b2_1, Wr, br):
    raise NotImplementedError("write your pallas kernel here")


# speedup vs baseline: 4.4084x; 4.4084x over previous
import functools

import jax
import jax.numpy as jnp
from jax import lax
from jax.experimental import pallas as pl
from jax.experimental.pallas import tpu as pltpu
from jax.experimental.pallas import tpu_sc as plsc

N_NODES = 10000
N_EDGES = 320000
D = 128

NC = 2
NS = 16
CH = 128
CPT = 79
EDGES_PER_TILE = CPT * CH
E_PAD = NC * NS * EDGES_PER_TILE
N_PAD = 10112
ROWS_PER_TILE = N_PAD // NS


@functools.cache
def _make_agg_kernel():
    mesh = plsc.VectorSubcoreMesh(core_axis_name="c", subcore_axis_name="s")

    @functools.partial(
        pl.kernel,
        mesh=mesh,
        out_type=jax.ShapeDtypeStruct((NC, N_PAD, D), jnp.float32),
        scratch_types=[
            pltpu.VMEM((CPT, CH), jnp.int32),
            pltpu.VMEM((CPT, CH), jnp.int32),
            pltpu.VMEM((CH, D), jnp.float32),
            pltpu.VMEM_SHARED((N_PAD, D), jnp.float32),
            pltpu.SemaphoreType.DMA,
        ],
    )
    def agg(h_hbm, src_hbm, dst_hbm, out_hbm, src_v, dst_v, rows_v, acc, sem):
        c = lax.axis_index("c")
        s = lax.axis_index("s")

        pltpu.sync_copy(src_hbm.at[c, s], src_v)
        pltpu.sync_copy(dst_hbm.at[c, s], dst_v)

        def zrow(r, carry):
            for k in range(D // 16):
                rows_v[r, pl.ds(k * 16, 16)] = jnp.zeros((16,), jnp.float32)
            return carry

        lax.fori_loop(0, CH, zrow, 0)
        base = s * ROWS_PER_TILE
        full = ROWS_PER_TILE // CH
        rem = ROWS_PER_TILE - full * CH
        for k in range(full):
            pltpu.sync_copy(rows_v, acc.at[pl.ds(base + k * CH, CH)])
        if rem:
            pltpu.sync_copy(rows_v.at[pl.ds(0, rem)],
                            acc.at[pl.ds(base + full * CH, rem)])
        plsc.subcore_barrier()

        def chunk(j, carry):
            pltpu.async_copy(h_hbm.at[src_v.at[j]], rows_v, sem).wait()
            pltpu.sync_copy(rows_v, acc.at[dst_v.at[j]], add=True)
            return carry

        lax.fori_loop(0, CPT, chunk, 0)
        plsc.subcore_barrier()

        pltpu.sync_copy(acc.at[pl.ds(base, ROWS_PER_TILE)],
                        out_hbm.at[c, pl.ds(base, ROWS_PER_TILE)])

    return agg


_ROW_BLK = 1000


def _mlp1_body(h_ref, p_ref, w1_ref, b1_ref, w2_ref, b2_ref, o_ref):
    z = h_ref[...] + p_ref[0] + p_ref[1]
    a = jnp.dot(z, w1_ref[...], preferred_element_type=jnp.float32) + b1_ref[...]
    a = jnp.maximum(a, 0.0)
    z2 = jnp.dot(a, w2_ref[...], preferred_element_type=jnp.float32) + b2_ref[...]
    o_ref[...] = jnp.maximum(z2, 0.0)


def _mlp2_body(h_ref, p_ref, w1_ref, b1_ref, w2_ref, b2_ref, wr_ref, br_ref,
               o_ref):
    z = h_ref[...] + p_ref[0] + p_ref[1]
    a = jnp.dot(z, w1_ref[...], preferred_element_type=jnp.float32) + b1_ref[...]
    a = jnp.maximum(a, 0.0)
    z2 = jnp.dot(a, w2_ref[...], preferred_element_type=jnp.float32) + b2_ref[...]
    h2 = jnp.maximum(z2, 0.0)
    o_ref[...] = jnp.dot(h2, wr_ref[...], preferred_element_type=jnp.float32) + br_ref[...]


def _row_spec():
    return pl.BlockSpec((_ROW_BLK, D), lambda i: (i, 0))


def _part_spec():
    return pl.BlockSpec((2, _ROW_BLK, D), lambda i: (0, i, 0))


def _full_spec(shape):
    return pl.BlockSpec(shape, lambda i: tuple(0 for _ in shape))


def _mlp1(h, p, w1, b1, w2, b2):
    return pl.pallas_call(
        _mlp1_body,
        grid=(N_NODES // _ROW_BLK,),
        in_specs=[
            _row_spec(), _part_spec(),
            _full_spec((D, D)), _full_spec((1, D)),
            _full_spec((D, D)), _full_spec((1, D)),
        ],
        out_specs=_row_spec(),
        out_shape=jax.ShapeDtypeStruct((N_NODES, D), jnp.float32),
    )(h, p, w1, b1, w2, b2)


def _mlp2(h, p, w1, b1, w2, b2, wr, br):
    return pl.pallas_call(
        _mlp2_body,
        grid=(N_NODES // _ROW_BLK,),
        in_specs=[
            _row_spec(), _part_spec(),
            _full_spec((D, D)), _full_spec((1, D)),
            _full_spec((D, D)), _full_spec((1, D)),
            _full_spec((D, 1)), _full_spec((1, 1)),
        ],
        out_specs=pl.BlockSpec((_ROW_BLK, 1), lambda i: (i, 0)),
        out_shape=jax.ShapeDtypeStruct((N_NODES, 1), jnp.float32),
    )(h, p, w1, b1, w2, b2, wr, br)


@jax.jit
def kernel(x, edge_index, W1_0, b1_0, W2_0, b2_0, W1_1, b1_1, W2_1, b2_1, Wr, br):
    src = edge_index[0].astype(jnp.int32)
    dst = edge_index[1].astype(jnp.int32)
    pad = E_PAD - N_EDGES
    src_g = jnp.concatenate([src, jnp.zeros((pad,), jnp.int32)]).reshape(
        NC, NS, CPT, CH)
    dst_g = jnp.concatenate([dst, jnp.full((pad,), N_NODES, jnp.int32)]).reshape(
        NC, NS, CPT, CH)

    b1_0r = b1_0.reshape(1, D)
    b2_0r = b2_0.reshape(1, D)
    b1_1r = b1_1.reshape(1, D)
    b2_1r = b2_1.reshape(1, D)
    brr = br.reshape(1, 1)

    agg = _make_agg_kernel()
    p0 = agg(x, src_g, dst_g)[:, :N_NODES, :]
    h1 = _mlp1(x, p0, W1_0, b1_0r, W2_0, b2_0r)
    p1 = agg(h1, src_g, dst_g)[:, :N_NODES, :]
    out = _mlp2(h1, p1, W1_1, b1_1r, W2_1, b2_1r, Wr, brr)
    return out
